# Initial kernel scaffold; baseline (speedup 1.0000x reference)
import jax
import jax.numpy as jnp
from jax.experimental import pallas as pl


def kernel(predicted_locs, predicted_scores, boxes, priors):
    raise NotImplementedError("write your pallas kernel here")


# speedup vs baseline: 59.9363x; 59.9363x over previous
import jax
import jax.numpy as jnp
from jax.experimental import pallas as pl
from jax.experimental.pallas import tpu as pltpu

_NOBJ = 8
_THRESHOLD = 0.5
_NEG_POS_RATIO = 3
_MAXK = _NEG_POS_RATIO * _NOBJ
_NEG_INF = -1e30
_ROWS = 69
_LANES = 128


def _loss_kernel(s0_ref, s1_ref, priorsb_ref, priors8_ref, boxes_ref,
                 locs8_ref, scores8_ref, out_ref):
    b = pl.program_id(0)
    row_i = jax.lax.broadcasted_iota(jnp.int32, (_ROWS, _LANES), 0)
    col_i = jax.lax.broadcasted_iota(jnp.int32, (_ROWS, _LANES), 1)
    row = row_i.astype(jnp.float32)
    col = col_i.astype(jnp.float32)
    flat = (row_i * _LANES + col_i).astype(jnp.float32)
    valid = flat < 8732.0

    pcx = priorsb_ref[0]
    pcy = priorsb_ref[1]
    pw = priorsb_ref[2]
    ph = priorsb_ref[3]
    px0 = pcx - pw * 0.5
    py0 = pcy - ph * 0.5
    px1 = pcx + pw * 0.5
    py1 = pcy + ph * 0.5
    parea = (px1 - px0) * (py1 - py0)

    m = jnp.full((_ROWS, _LANES), -1.0, dtype=jnp.float32)
    amax = jnp.zeros((_ROWS, _LANES), dtype=jnp.float32)
    pfe = []
    for j in range(_NOBJ):
        bcx = boxes_ref[0, j, 0]
        bcy = boxes_ref[0, j, 1]
        bw = boxes_ref[0, j, 2]
        bh = boxes_ref[0, j, 3]
        bx0 = bcx - bw * 0.5
        by0 = bcy - bh * 0.5
        bx1 = bcx + bw * 0.5
        by1 = bcy + bh * 0.5
        barea = (bx1 - bx0) * (by1 - by0)
        iw = jnp.maximum(jnp.minimum(px1, bx1) - jnp.maximum(px0, bx0), 0.0)
        ih = jnp.maximum(jnp.minimum(py1, by1) - jnp.maximum(py0, by0), 0.0)
        inter = iw * ih
        iou = inter / (parea + barea - inter)
        upd = iou > m
        amax = jnp.where(upd, float(j), amax)
        m = jnp.where(upd, iou, m)
        mx = jnp.max(jnp.where(valid, iou, -1.0))
        pfe.append(jnp.min(jnp.where(valid & (iou == mx), flat, 1e9)))

    for j in range(_NOBJ):
        hit = flat == pfe[j]
        amax = jnp.where(hit, float(j), amax)
        m = jnp.where(hit, 1.0, m)

    pres = [jnp.max(jnp.where(valid & (amax == float(j)), 1.0, 0.0))
            for j in range(_NOBJ)]

    lcol = col[0:1, :]
    o_lane = amax[0:1, :]
    m_lane = m[0:1, :]
    pres_lane = jnp.zeros((1, _LANES), dtype=jnp.float32)
    for j in range(_NOBJ):
        pres_lane = jnp.where(lcol == float(j), pres[j], pres_lane)
    pos_lane = (lcol < float(_NOBJ)) & (m_lane >= _THRESHOLD) \
        & (pres_lane > 0.5)
    n_pos = jnp.sum(jnp.where(pos_lane, 1.0, 0.0))

    bcx_l = jnp.ones((1, _LANES), dtype=jnp.float32)
    bcy_l = jnp.ones((1, _LANES), dtype=jnp.float32)
    bw_l = jnp.ones((1, _LANES), dtype=jnp.float32)
    bh_l = jnp.ones((1, _LANES), dtype=jnp.float32)
    for j in range(_NOBJ):
        sel = o_lane == float(j)
        bcx_l = jnp.where(sel, boxes_ref[0, j, 0], bcx_l)
        bcy_l = jnp.where(sel, boxes_ref[0, j, 1], bcy_l)
        bw_l = jnp.where(sel, boxes_ref[0, j, 2], bw_l)
        bh_l = jnp.where(sel, boxes_ref[0, j, 3], bh_l)

    p8 = priors8_ref
    pcx_l = jnp.zeros((1, _LANES), dtype=jnp.float32)
    pcy_l = jnp.zeros((1, _LANES), dtype=jnp.float32)
    pw_l = jnp.ones((1, _LANES), dtype=jnp.float32)
    ph_l = jnp.ones((1, _LANES), dtype=jnp.float32)
    lx = [jnp.zeros((1, _LANES), dtype=jnp.float32) for _ in range(4)]
    s0_l = jnp.zeros((1, _LANES), dtype=jnp.float32)
    s1_l = jnp.zeros((1, _LANES), dtype=jnp.float32)
    for p in range(_NOBJ):
        sel = lcol == float(p)
        pcx_l = jnp.where(sel, p8[p, 0], pcx_l)
        pcy_l = jnp.where(sel, p8[p, 1], pcy_l)
        pw_l = jnp.where(sel, p8[p, 2], pw_l)
        ph_l = jnp.where(sel, p8[p, 3], ph_l)
        for k in range(4):
            lx[k] = jnp.where(sel, locs8_ref[0, p, k], lx[k])
        s0_l = jnp.where(sel, scores8_ref[0, p, 0], s0_l)
        s1_l = jnp.where(sel, scores8_ref[0, p, 1], s1_l)

    gcx = (bcx_l - pcx_l) / (pw_l / 10.0)
    gcy = (bcy_l - pcy_l) / (ph_l / 10.0)
    gw = jnp.log(bw_l / pw_l) * 5.0
    gh = jnp.log(bh_l / ph_l) * 5.0
    loc_abs = (jnp.abs(lx[0] - gcx) + jnp.abs(lx[1] - gcy)
               + jnp.abs(lx[2] - gw) + jnp.abs(lx[3] - gh))
    loc_num = jnp.sum(jnp.where(pos_lane, loc_abs, 0.0))

    smax = jnp.maximum(s0_l, s1_l)
    smin = jnp.minimum(s0_l, s1_l)
    lse = smax + jnp.log(1.0 + jnp.exp(smin - smax))
    conf_pos = jnp.sum(jnp.where(pos_lane, lse - s1_l, 0.0))

    d = s1_ref[0] - s0_ref[0]
    posmask = (row == 0.0) & jnp.broadcast_to(pos_lane, (_ROWS, _LANES))
    v = jnp.where(valid & (~posmask), d, _NEG_INF)
    tops = jnp.full((1, _LANES), _NEG_INF, dtype=jnp.float32)
    for i in range(_MAXK):
        mi = jnp.max(v)
        idx = jnp.min(jnp.where(v == mi, flat, 1e9))
        v = jnp.where(flat == idx, _NEG_INF, v)
        tops = jnp.where(lcol == float(i), mi, tops)
    tmax = jnp.maximum(tops, 0.0)
    tneg = -jnp.abs(tops)
    closs = tmax + jnp.log(1.0 + jnp.exp(tneg))
    hard_ok = (lcol < float(_NEG_POS_RATIO) * n_pos) & (lcol < float(_MAXK))
    hard_sum = jnp.sum(jnp.where(hard_ok, closs, 0.0))

    contrib = (jnp.where(lcol == 0.0, n_pos, 0.0)
               + jnp.where(lcol == 1.0, loc_num, 0.0)
               + jnp.where(lcol == 2.0, conf_pos, 0.0)
               + jnp.where(lcol == 3.0, hard_sum, 0.0))

    @pl.when(b == 0)
    def _init():
        out_ref[...] = jnp.zeros_like(out_ref)

    out_ref[...] += contrib


def kernel(predicted_locs, predicted_scores, boxes, priors):
    batch, n_priors, _ = predicted_locs.shape
    pad = _ROWS * _LANES - n_priors

    s0 = predicted_scores[..., 0]
    s1 = predicted_scores[..., 1]
    s0p = jnp.pad(s0, ((0, 0), (0, pad))).reshape(batch, _ROWS, _LANES)
    s1p = jnp.pad(s1, ((0, 0), (0, pad))).reshape(batch, _ROWS, _LANES)

    pad_prior = jnp.tile(
        jnp.asarray([[-100.0, -100.0, 1.0, 1.0]], dtype=jnp.float32),
        (pad, 1))
    priorsb = jnp.concatenate([priors, pad_prior], axis=0).T.reshape(
        4, _ROWS, _LANES)
    priors8 = priors[:_NOBJ]
    locs8 = predicted_locs[:, :_NOBJ, :]
    scores8 = predicted_scores[:, :_NOBJ, :]

    out = pl.pallas_call(
        _loss_kernel,
        grid=(batch,),
        in_specs=[
            pl.BlockSpec((1, _ROWS, _LANES), lambda b: (b, 0, 0)),
            pl.BlockSpec((1, _ROWS, _LANES), lambda b: (b, 0, 0)),
            pl.BlockSpec((4, _ROWS, _LANES), lambda b: (0, 0, 0)),
            pl.BlockSpec((_NOBJ, 4), lambda b: (0, 0),
                         memory_space=pltpu.SMEM),
            pl.BlockSpec((1, _NOBJ, 4), lambda b: (b, 0, 0),
                         memory_space=pltpu.SMEM),
            pl.BlockSpec((1, _NOBJ, 4), lambda b: (b, 0, 0),
                         memory_space=pltpu.SMEM),
            pl.BlockSpec((1, _NOBJ, 2), lambda b: (b, 0, 0),
                         memory_space=pltpu.SMEM),
        ],
        out_specs=pl.BlockSpec((1, _LANES), lambda b: (0, 0)),
        out_shape=jax.ShapeDtypeStruct((1, _LANES), jnp.float32),
    )(s0p, s1p, priorsb, priors8, boxes, locs8, scores8)

    n_pos_total = out[0, 0]
    loc_loss = out[0, 1] / (n_pos_total * 4.0)
    conf_loss = (out[0, 2] + out[0, 3]) / n_pos_total
    return conf_loss + loc_loss


# speedup vs baseline: 99.5499x; 1.6609x over previous
import jax
import jax.numpy as jnp
from jax.experimental import pallas as pl
from jax.experimental.pallas import tpu as pltpu

_NOBJ = 8
_THRESHOLD = 0.5
_NEG_POS_RATIO = 3
_MAXK = _NEG_POS_RATIO * _NOBJ
_NEG_INF = -1e30
_ROWS = 69
_LANES = 128
_IMGB = 8


def _one_image(i, s0_ref, s1_ref, boxes_ref, locs8_ref, scores8_ref,
               prior_geom, iotas):
    (px0, py0, px1, py1, parea, pcx_l, pcy_l, pw_l, ph_l) = prior_geom
    (row, col, flat, valid, lcol) = iotas

    m = jnp.full((_ROWS, _LANES), -1.0, dtype=jnp.float32)
    amax = jnp.zeros((_ROWS, _LANES), dtype=jnp.float32)
    pfe = []
    for j in range(_NOBJ):
        bcx = boxes_ref[i, j, 0]
        bcy = boxes_ref[i, j, 1]
        bw = boxes_ref[i, j, 2]
        bh = boxes_ref[i, j, 3]
        bx0 = bcx - bw * 0.5
        by0 = bcy - bh * 0.5
        bx1 = bcx + bw * 0.5
        by1 = bcy + bh * 0.5
        barea = (bx1 - bx0) * (by1 - by0)
        iw = jnp.maximum(jnp.minimum(px1, bx1) - jnp.maximum(px0, bx0), 0.0)
        ih = jnp.maximum(jnp.minimum(py1, by1) - jnp.maximum(py0, by0), 0.0)
        inter = iw * ih
        iou = inter / (parea + barea - inter)
        upd = iou > m
        amax = jnp.where(upd, float(j), amax)
        m = jnp.where(upd, iou, m)
        mx = jnp.max(jnp.where(valid, iou, -1.0))
        pfe.append(jnp.min(jnp.where(valid & (iou == mx), flat, 1e9)))

    for j in range(_NOBJ):
        hit = flat == pfe[j]
        amax = jnp.where(hit, float(j), amax)
        m = jnp.where(hit, 1.0, m)

    pres = [jnp.max(jnp.where(valid & (amax == float(j)), 1.0, 0.0))
            for j in range(_NOBJ)]

    o_lane = amax[0:1, :]
    m_lane = m[0:1, :]
    pres_lane = jnp.zeros((1, _LANES), dtype=jnp.float32)
    for j in range(_NOBJ):
        pres_lane = jnp.where(lcol == float(j), pres[j], pres_lane)
    pos_lane = (lcol < float(_NOBJ)) & (m_lane >= _THRESHOLD) \
        & (pres_lane > 0.5)
    n_pos = jnp.sum(jnp.where(pos_lane, 1.0, 0.0))

    bcx_l = jnp.ones((1, _LANES), dtype=jnp.float32)
    bcy_l = jnp.ones((1, _LANES), dtype=jnp.float32)
    bw_l = jnp.ones((1, _LANES), dtype=jnp.float32)
    bh_l = jnp.ones((1, _LANES), dtype=jnp.float32)
    for j in range(_NOBJ):
        sel = o_lane == float(j)
        bcx_l = jnp.where(sel, boxes_ref[i, j, 0], bcx_l)
        bcy_l = jnp.where(sel, boxes_ref[i, j, 1], bcy_l)
        bw_l = jnp.where(sel, boxes_ref[i, j, 2], bw_l)
        bh_l = jnp.where(sel, boxes_ref[i, j, 3], bh_l)

    lx = [jnp.zeros((1, _LANES), dtype=jnp.float32) for _ in range(4)]
    s0_l = jnp.zeros((1, _LANES), dtype=jnp.float32)
    s1_l = jnp.zeros((1, _LANES), dtype=jnp.float32)
    for p in range(_NOBJ):
        sel = lcol == float(p)
        for k in range(4):
            lx[k] = jnp.where(sel, locs8_ref[i, p, k], lx[k])
        s0_l = jnp.where(sel, scores8_ref[i, p, 0], s0_l)
        s1_l = jnp.where(sel, scores8_ref[i, p, 1], s1_l)

    gcx = (bcx_l - pcx_l) / (pw_l / 10.0)
    gcy = (bcy_l - pcy_l) / (ph_l / 10.0)
    gw = jnp.log(bw_l / pw_l) * 5.0
    gh = jnp.log(bh_l / ph_l) * 5.0
    loc_abs = (jnp.abs(lx[0] - gcx) + jnp.abs(lx[1] - gcy)
               + jnp.abs(lx[2] - gw) + jnp.abs(lx[3] - gh))
    loc_num = jnp.sum(jnp.where(pos_lane, loc_abs, 0.0))

    smax = jnp.maximum(s0_l, s1_l)
    smin = jnp.minimum(s0_l, s1_l)
    lse = smax + jnp.log(1.0 + jnp.exp(smin - smax))
    conf_pos = jnp.sum(jnp.where(pos_lane, lse - s1_l, 0.0))

    d = s1_ref[i] - s0_ref[i]
    posmask = (row == 0.0) & jnp.broadcast_to(pos_lane, (_ROWS, _LANES))
    v = jnp.where(valid & (~posmask), d, _NEG_INF)
    return v, n_pos, loc_num, conf_pos


def _loss_kernel(s0_ref, s1_ref, priorsb_ref, priors8_ref, boxes_ref,
                 locs8_ref, scores8_ref, out_ref):
    b = pl.program_id(0)

    row_i = jax.lax.broadcasted_iota(jnp.int32, (_ROWS, _LANES), 0)
    col_i = jax.lax.broadcasted_iota(jnp.int32, (_ROWS, _LANES), 1)
    row = row_i.astype(jnp.float32)
    col = col_i.astype(jnp.float32)
    flat = (row_i * _LANES + col_i).astype(jnp.float32)
    valid = flat < 8732.0
    lcol = col[0:1, :]

    pcx = priorsb_ref[0]
    pcy = priorsb_ref[1]
    pw = priorsb_ref[2]
    ph = priorsb_ref[3]
    px0 = pcx - pw * 0.5
    py0 = pcy - ph * 0.5
    px1 = pcx + pw * 0.5
    py1 = pcy + ph * 0.5
    parea = (px1 - px0) * (py1 - py0)

    p8 = priors8_ref
    pcx_l = jnp.zeros((1, _LANES), dtype=jnp.float32)
    pcy_l = jnp.zeros((1, _LANES), dtype=jnp.float32)
    pw_l = jnp.ones((1, _LANES), dtype=jnp.float32)
    ph_l = jnp.ones((1, _LANES), dtype=jnp.float32)
    for p in range(_NOBJ):
        sel = lcol == float(p)
        pcx_l = jnp.where(sel, p8[p, 0], pcx_l)
        pcy_l = jnp.where(sel, p8[p, 1], pcy_l)
        pw_l = jnp.where(sel, p8[p, 2], pw_l)
        ph_l = jnp.where(sel, p8[p, 3], ph_l)

    prior_geom = (px0, py0, px1, py1, parea, pcx_l, pcy_l, pw_l, ph_l)
    iotas = (row, col, flat, valid, lcol)

    vs, n_poss, loc_nums, conf_poss = [], [], [], []
    for i in range(_IMGB):
        v, n_pos, loc_num, conf_pos = _one_image(
            i, s0_ref, s1_ref, boxes_ref, locs8_ref, scores8_ref,
            prior_geom, iotas)
        vs.append(v)
        n_poss.append(n_pos)
        loc_nums.append(loc_num)
        conf_poss.append(conf_pos)

    topss = [jnp.full((1, _LANES), _NEG_INF, dtype=jnp.float32)
             for _ in range(_IMGB)]
    for k in range(_MAXK):
        for i in range(_IMGB):
            mi = jnp.max(vs[i])
            idx = jnp.min(jnp.where(vs[i] == mi, flat, 1e9))
            vs[i] = jnp.where(flat == idx, _NEG_INF, vs[i])
            topss[i] = jnp.where(lcol == float(k), mi, topss[i])

    hard_sums = []
    for i in range(_IMGB):
        tops = topss[i]
        tmax = jnp.maximum(tops, 0.0)
        tneg = -jnp.abs(tops)
        closs = tmax + jnp.log(1.0 + jnp.exp(tneg))
        hard_ok = (lcol < float(_NEG_POS_RATIO) * n_poss[i]) \
            & (lcol < float(_MAXK))
        hard_sums.append(jnp.sum(jnp.where(hard_ok, closs, 0.0)))

    n_pos_t = n_poss[0]
    loc_t = loc_nums[0]
    cpos_t = conf_poss[0]
    hard_t = hard_sums[0]
    for i in range(1, _IMGB):
        n_pos_t += n_poss[i]
        loc_t += loc_nums[i]
        cpos_t += conf_poss[i]
        hard_t += hard_sums[i]

    contrib = (jnp.where(lcol == 0.0, n_pos_t, 0.0)
               + jnp.where(lcol == 1.0, loc_t, 0.0)
               + jnp.where(lcol == 2.0, cpos_t, 0.0)
               + jnp.where(lcol == 3.0, hard_t, 0.0))

    @pl.when(b == 0)
    def _init():
        out_ref[...] = jnp.zeros_like(out_ref)

    out_ref[...] += contrib


def kernel(predicted_locs, predicted_scores, boxes, priors):
    batch, n_priors, _ = predicted_locs.shape
    pad = _ROWS * _LANES - n_priors

    s0 = predicted_scores[..., 0]
    s1 = predicted_scores[..., 1]
    s0p = jnp.pad(s0, ((0, 0), (0, pad))).reshape(batch, _ROWS, _LANES)
    s1p = jnp.pad(s1, ((0, 0), (0, pad))).reshape(batch, _ROWS, _LANES)

    pad_prior = jnp.tile(
        jnp.asarray([[-100.0, -100.0, 1.0, 1.0]], dtype=jnp.float32),
        (pad, 1))
    priorsb = jnp.concatenate([priors, pad_prior], axis=0).T.reshape(
        4, _ROWS, _LANES)
    priors8 = priors[:_NOBJ]
    locs8 = predicted_locs[:, :_NOBJ, :]
    scores8 = predicted_scores[:, :_NOBJ, :]

    out = pl.pallas_call(
        _loss_kernel,
        grid=(batch // _IMGB,),
        in_specs=[
            pl.BlockSpec((_IMGB, _ROWS, _LANES), lambda b: (b, 0, 0)),
            pl.BlockSpec((_IMGB, _ROWS, _LANES), lambda b: (b, 0, 0)),
            pl.BlockSpec((4, _ROWS, _LANES), lambda b: (0, 0, 0)),
            pl.BlockSpec((_NOBJ, 4), lambda b: (0, 0),
                         memory_space=pltpu.SMEM),
            pl.BlockSpec((_IMGB, _NOBJ, 4), lambda b: (b, 0, 0),
                         memory_space=pltpu.SMEM),
            pl.BlockSpec((_IMGB, _NOBJ, 4), lambda b: (b, 0, 0),
                         memory_space=pltpu.SMEM),
            pl.BlockSpec((_IMGB, _NOBJ, 2), lambda b: (b, 0, 0),
                         memory_space=pltpu.SMEM),
        ],
        out_specs=pl.BlockSpec((1, _LANES), lambda b: (0, 0)),
        out_shape=jax.ShapeDtypeStruct((1, _LANES), jnp.float32),
    )(s0p, s1p, priorsb, priors8, boxes, locs8, scores8)

    n_pos_total = out[0, 0]
    loc_loss = out[0, 1] / (n_pos_total * 4.0)
    conf_loss = (out[0, 2] + out[0, 3]) / n_pos_total
    return conf_loss + loc_loss


# speedup vs baseline: 123.3468x; 1.2390x over previous
import functools

import jax
import jax.numpy as jnp
from jax import lax
from jax.experimental import pallas as pl
from jax.experimental.pallas import tpu as pltpu
from jax.experimental.pallas import tpu_sc as plsc

_NOBJ = 8
_THRESHOLD = 0.5
_NEG_POS_RATIO = 3
_NEG_INF = -1e30
_INIT_KEY = -3e30
_ROWS = 69
_LANES = 128
_IMGB = 8
_NPAD = _ROWS * _LANES
_SEGS = _NPAD // 16
_NCAND = 32
_IMGS_PER_TILE = 4


_STAGES_FULL = [(2, 1), (4, 2), (4, 1), (8, 4), (8, 2), (8, 1),
                (16, 8), (16, 4), (16, 2), (16, 1)]
_STAGES_MERGE = [(16, 8), (16, 4), (16, 2), (16, 1)]


def _mk_stage_consts(lane, stages):
    out = []
    for (k, j) in stages:
        p = lax.bitwise_xor(lane, j)
        low = jnp.where((lane & j) == 0, 1, 0)
        dirmax = jnp.where((lane & k) == 0, 1, 0)
        keep_max = 1 - lax.bitwise_xor(low, dirmax)
        out.append((p, keep_max.astype(jnp.float32)))
    return out


def _net(k_, v_, stage_consts):
    for (p, km) in stage_consts:
        pk = k_[p]
        pv = v_[p]
        agef = jnp.where(k_ >= pk, 1.0, 0.0)
        sel_a = km == agef
        k_ = jnp.where(sel_a, k_, pk)
        v_ = jnp.where(sel_a, v_, pv)
    return k_, v_


def _halves(rev, ak, av, bk, bv):
    rbk = bk[rev]
    rbv = bv[rev]
    sel = ak >= rbk
    hik = jnp.where(sel, ak, rbk)
    hiv = jnp.where(sel, av, rbv)
    lok = jnp.where(sel, rbk, ak)
    lov = jnp.where(sel, rbv, av)
    return hik, hiv, lok, lov


def _mine_kernel(s0_hbm, s1_hbm, outk_hbm, outi_hbm, b0, b1, db, dbi,
                 okv, oiv):
    cid = lax.axis_index("c")
    sid = lax.axis_index("s")
    wid = sid * 2 + cid
    lane = lax.broadcasted_iota(jnp.int32, (16,), 0)
    rev = 15 - lane
    sf = _mk_stage_consts(lane, _STAGES_FULL)
    sm = _mk_stage_consts(lane, _STAGES_MERGE)
    tperm = [lax.bitwise_xor(lane, sh) for sh in (8, 4, 2, 1)]

    def merge_c(t1k, t1v, t2k, t2v, ck, cv):
        sck, scv = _net(ck, cv, sf)
        uk, uv, ulk, ulv = _halves(rev, t2k, t2v, sck, scv)
        uk, uv = _net(uk, uv, sm)
        ulk, ulv = _net(ulk, ulv, sm)
        nt1k, nt1v, wk, wv = _halves(rev, t1k, t1v, uk, uv)
        nt1k, nt1v = _net(nt1k, nt1v, sm)
        wk, wv = _net(wk, wv, sm)
        nt2k, nt2v, _, _ = _halves(rev, wk, wv, ulk, ulv)
        nt2k, nt2v = _net(nt2k, nt2v, sm)
        return nt1k, nt1v, nt2k, nt2v

    for g in range(_IMGS_PER_TILE):
        img = wid * _IMGS_PER_TILE + g
        pltpu.sync_copy(s0_hbm.at[pl.ds(img * _NPAD, _NPAD)], b0)
        pltpu.sync_copy(s1_hbm.at[pl.ds(img * _NPAD, _NPAD)], b1)
        tail = b1[pl.ds(8720, 16)]
        lanef = lane.astype(jnp.float32)
        b1[pl.ds(8720, 16)] = jnp.where(lanef < 11.5, tail, _NEG_INF)
        for t in range(546, _SEGS):
            b1[pl.ds(t * 16, 16)] = jnp.full((16,), _NEG_INF, jnp.float32)

        def pass1(j, carry):
            m1, m2 = carry
            base = j * 16
            c = b1[pl.ds(base, 16)] - b0[pl.ds(base, 16)]
            db[pl.ds(base, 16)] = c
            m2n = jnp.maximum(m2, jnp.minimum(m1, c))
            m1n = jnp.maximum(m1, c)
            return m1n, m2n

        m1, m2 = lax.fori_loop(
            0, _SEGS, pass1,
            (jnp.full((16,), _INIT_KEY, jnp.float32),
             jnp.full((16,), _INIT_KEY, jnp.float32)))
        tv = jnp.minimum(m1, m2)
        for p in tperm:
            tv = jnp.minimum(tv, tv[p])
        thr = tv[0]

        def pass2b(j, off):
            base = j * 16
            c = db[pl.ds(base, 16)]
            okv2_base = off * 16
            dbi[pl.ds(okv2_base, 16)] = (lane + base).astype(jnp.float32)
            b0[pl.ds(okv2_base, 16)] = c
            mx = c
            for p in tperm:
                mx = jnp.maximum(mx, mx[p])
            return off + jnp.where(mx[0] >= thr, 1, 0)

        nsv = lax.fori_loop(0, _SEGS, pass2b, jnp.int32(0))

        def fin(j, carry):
            t1k, t1v, t2k, t2v = carry
            base = j * 16
            ck = b0[pl.ds(base, 16)]
            cv = dbi[pl.ds(base, 16)]
            return merge_c(t1k, t1v, t2k, t2v, ck, cv)

        init = (jnp.full((16,), _INIT_KEY, jnp.float32),
                jnp.zeros((16,), jnp.float32),
                jnp.full((16,), _INIT_KEY, jnp.float32),
                jnp.zeros((16,), jnp.float32))
        t1k, t1v, t2k, t2v = lax.fori_loop(0, nsv, fin, init)

        okv[pl.ds(0, 16)] = t1k
        okv[pl.ds(16, 16)] = t2k
        oiv[pl.ds(0, 16)] = t1v.astype(jnp.int32)
        oiv[pl.ds(16, 16)] = t2v.astype(jnp.int32)
        pltpu.sync_copy(okv, outk_hbm.at[pl.ds(img * _NCAND, _NCAND)])
        pltpu.sync_copy(oiv, outi_hbm.at[pl.ds(img * _NCAND, _NCAND)])


def _mine(s0f, s1f, batch):
    mesh = plsc.VectorSubcoreMesh(core_axis_name="c", subcore_axis_name="s")
    f = pl.kernel(
        _mine_kernel,
        out_type=[
            jax.ShapeDtypeStruct((batch * _NCAND,), jnp.float32),
            jax.ShapeDtypeStruct((batch * _NCAND,), jnp.int32),
        ],
        mesh=mesh,
        scratch_types=[
            pltpu.VMEM((_NPAD,), jnp.float32),
            pltpu.VMEM((_NPAD,), jnp.float32),
            pltpu.VMEM((_NPAD,), jnp.float32),
            pltpu.VMEM((_NPAD,), jnp.float32),
            pltpu.VMEM((_NCAND,), jnp.float32),
            pltpu.VMEM((_NCAND,), jnp.int32),
        ],
    )
    return f(s0f, s1f)


def _one_image(i, boxes_ref, locs8_ref, scores8_ref, cank_ref, cani_ref,
               prior_geom, iotas):
    (px0, py0, px1, py1, parea, pcx_l, pcy_l, pw_l, ph_l) = prior_geom
    (row, col, flat, valid, lcol, lcol32) = iotas

    m = jnp.full((_ROWS, _LANES), -1.0, dtype=jnp.float32)
    amax = jnp.zeros((_ROWS, _LANES), dtype=jnp.float32)
    pfe = []
    for j in range(_NOBJ):
        bcx = boxes_ref[i, j, 0]
        bcy = boxes_ref[i, j, 1]
        bw = boxes_ref[i, j, 2]
        bh = boxes_ref[i, j, 3]
        bx0 = bcx - bw * 0.5
        by0 = bcy - bh * 0.5
        bx1 = bcx + bw * 0.5
        by1 = bcy + bh * 0.5
        barea = (bx1 - bx0) * (by1 - by0)
        iw = jnp.maximum(jnp.minimum(px1, bx1) - jnp.maximum(px0, bx0), 0.0)
        ih = jnp.maximum(jnp.minimum(py1, by1) - jnp.maximum(py0, by0), 0.0)
        inter = iw * ih
        iou = inter / (parea + barea - inter)
        upd = iou > m
        amax = jnp.where(upd, float(j), amax)
        m = jnp.where(upd, iou, m)
        mx = jnp.max(jnp.where(valid, iou, -1.0))
        pfe.append(jnp.min(jnp.where(valid & (iou == mx), flat, 1e9)))

    for j in range(_NOBJ):
        hit = flat == pfe[j]
        amax = jnp.where(hit, float(j), amax)
        m = jnp.where(hit, 1.0, m)

    pres = [jnp.max(jnp.where(valid & (amax == float(j)), 1.0, 0.0))
            for j in range(_NOBJ)]

    o_lane = amax[0:1, :]
    m_lane = m[0:1, :]
    pres_lane = jnp.zeros((1, _LANES), dtype=jnp.float32)
    for j in range(_NOBJ):
        pres_lane = jnp.where(lcol == float(j), pres[j], pres_lane)
    pos_lane = (lcol < float(_NOBJ)) & (m_lane >= _THRESHOLD) \
        & (pres_lane > 0.5)
    n_pos = jnp.sum(jnp.where(pos_lane, 1.0, 0.0))

    bcx_l = jnp.ones((1, _LANES), dtype=jnp.float32)
    bcy_l = jnp.ones((1, _LANES), dtype=jnp.float32)
    bw_l = jnp.ones((1, _LANES), dtype=jnp.float32)
    bh_l = jnp.ones((1, _LANES), dtype=jnp.float32)
    for j in range(_NOBJ):
        sel = o_lane == float(j)
        bcx_l = jnp.where(sel, boxes_ref[i, j, 0], bcx_l)
        bcy_l = jnp.where(sel, boxes_ref[i, j, 1], bcy_l)
        bw_l = jnp.where(sel, boxes_ref[i, j, 2], bw_l)
        bh_l = jnp.where(sel, boxes_ref[i, j, 3], bh_l)

    lx = [jnp.zeros((1, _LANES), dtype=jnp.float32) for _ in range(4)]
    s0_l = jnp.zeros((1, _LANES), dtype=jnp.float32)
    s1_l = jnp.zeros((1, _LANES), dtype=jnp.float32)
    for p in range(_NOBJ):
        sel = lcol == float(p)
        for k in range(4):
            lx[k] = jnp.where(sel, locs8_ref[i, p, k], lx[k])
        s0_l = jnp.where(sel, scores8_ref[i, p, 0], s0_l)
        s1_l = jnp.where(sel, scores8_ref[i, p, 1], s1_l)

    gcx = (bcx_l - pcx_l) / (pw_l / 10.0)
    gcy = (bcy_l - pcy_l) / (ph_l / 10.0)
    gw = jnp.log(bw_l / pw_l) * 5.0
    gh = jnp.log(bh_l / ph_l) * 5.0
    loc_abs = (jnp.abs(lx[0] - gcx) + jnp.abs(lx[1] - gcy)
               + jnp.abs(lx[2] - gw) + jnp.abs(lx[3] - gh))
    loc_num = jnp.sum(jnp.where(pos_lane, loc_abs, 0.0))

    smax = jnp.maximum(s0_l, s1_l)
    smin = jnp.minimum(s0_l, s1_l)
    lse = smax + jnp.log(1.0 + jnp.exp(smin - smax))
    conf_pos = jnp.sum(jnp.where(pos_lane, lse - s1_l, 0.0))

    pw2 = jnp.zeros((1, _LANES), dtype=jnp.float32)
    for p in range(_NOBJ):
        pw2 = jnp.where(lcol == float(p), float(2 ** p), pw2)
    enc = jnp.sum(jnp.where(pos_lane, pw2, 0.0))

    keys = cank_ref[i:i + 1, :]
    idxf = cani_ref[i:i + 1, :].astype(jnp.float32)
    drop = idxf < -1.0
    for p in range(_NOBJ):
        bit = jnp.floor(enc / float(2 ** p)) % 2.0
        drop = drop | ((idxf == float(p)) & (bit > 0.5))
    surv = ~drop
    run = jnp.where(surv, 1.0, 0.0)
    for sh in (1, 2, 4, 8, 16):
        shifted = jnp.where(lcol32 >= float(sh),
                            pltpu.roll(run, sh, axis=1), 0.0)
        run = run + shifted
    pick = surv & (run <= float(_NEG_POS_RATIO) * n_pos)
    tmax = jnp.maximum(keys, 0.0)
    closs = tmax + jnp.log(1.0 + jnp.exp(-jnp.abs(keys)))
    hard_sum = jnp.sum(jnp.where(pick, closs, 0.0))

    return n_pos, loc_num, conf_pos, hard_sum


def _loss_kernel(priorsb_ref, priors8_ref, boxes_ref, locs8_ref, scores8_ref,
                 cank_ref, cani_ref, out_ref):
    b = pl.program_id(0)

    row_i = jax.lax.broadcasted_iota(jnp.int32, (_ROWS, _LANES), 0)
    col_i = jax.lax.broadcasted_iota(jnp.int32, (_ROWS, _LANES), 1)
    row = row_i.astype(jnp.float32)
    col = col_i.astype(jnp.float32)
    flat = (row_i * _LANES + col_i).astype(jnp.float32)
    valid = flat < 8732.0
    lcol = col[0:1, :]
    lcol32 = jax.lax.broadcasted_iota(jnp.int32, (1, _NCAND), 1) \
        .astype(jnp.float32)

    pcx = priorsb_ref[0]
    pcy = priorsb_ref[1]
    pw = priorsb_ref[2]
    ph = priorsb_ref[3]
    px0 = pcx - pw * 0.5
    py0 = pcy - ph * 0.5
    px1 = pcx + pw * 0.5
    py1 = pcy + ph * 0.5
    parea = (px1 - px0) * (py1 - py0)

    p8 = priors8_ref
    pcx_l = jnp.zeros((1, _LANES), dtype=jnp.float32)
    pcy_l = jnp.zeros((1, _LANES), dtype=jnp.float32)
    pw_l = jnp.ones((1, _LANES), dtype=jnp.float32)
    ph_l = jnp.ones((1, _LANES), dtype=jnp.float32)
    for p in range(_NOBJ):
        sel = lcol == float(p)
        pcx_l = jnp.where(sel, p8[p, 0], pcx_l)
        pcy_l = jnp.where(sel, p8[p, 1], pcy_l)
        pw_l = jnp.where(sel, p8[p, 2], pw_l)
        ph_l = jnp.where(sel, p8[p, 3], ph_l)

    prior_geom = (px0, py0, px1, py1, parea, pcx_l, pcy_l, pw_l, ph_l)
    iotas = (row, col, flat, valid, lcol, lcol32)

    n_pos_t = jnp.float32(0.0)
    loc_t = jnp.float32(0.0)
    cpos_t = jnp.float32(0.0)
    hard_t = jnp.float32(0.0)
    for i in range(_IMGB):
        n_pos, loc_num, conf_pos, hard_sum = _one_image(
            i, boxes_ref, locs8_ref, scores8_ref, cank_ref, cani_ref,
            prior_geom, iotas)
        n_pos_t += n_pos
        loc_t += loc_num
        cpos_t += conf_pos
        hard_t += hard_sum

    contrib = (jnp.where(lcol == 0.0, n_pos_t, 0.0)
               + jnp.where(lcol == 1.0, loc_t, 0.0)
               + jnp.where(lcol == 2.0, cpos_t, 0.0)
               + jnp.where(lcol == 3.0, hard_t, 0.0))

    @pl.when(b == 0)
    def _init():
        out_ref[...] = jnp.zeros_like(out_ref)

    out_ref[...] += contrib


def kernel(predicted_locs, predicted_scores, boxes, priors):
    batch, n_priors, _ = predicted_locs.shape
    pad = _NPAD - n_priors

    s0 = predicted_scores[..., 0]
    s1 = predicted_scores[..., 1]
    s0f = jnp.pad(s0, ((0, 0), (0, pad))).reshape(batch * _NPAD)
    s1f = jnp.pad(s1, ((0, 0), (0, pad))).reshape(batch * _NPAD)

    outk, outi = _mine(s0f, s1f, batch)
    cank = outk.reshape(batch, _NCAND)
    cani = outi.reshape(batch, _NCAND)

    pad_prior = jnp.tile(
        jnp.asarray([[-100.0, -100.0, 1.0, 1.0]], dtype=jnp.float32),
        (pad, 1))
    priorsb = jnp.concatenate([priors, pad_prior], axis=0).T.reshape(
        4, _ROWS, _LANES)
    priors8 = priors[:_NOBJ]
    locs8 = predicted_locs[:, :_NOBJ, :]
    scores8 = predicted_scores[:, :_NOBJ, :]

    out = pl.pallas_call(
        _loss_kernel,
        grid=(batch // _IMGB,),
        in_specs=[
            pl.BlockSpec((4, _ROWS, _LANES), lambda b: (0, 0, 0)),
            pl.BlockSpec((_NOBJ, 4), lambda b: (0, 0),
                         memory_space=pltpu.SMEM),
            pl.BlockSpec((_IMGB, _NOBJ, 4), lambda b: (b, 0, 0),
                         memory_space=pltpu.SMEM),
            pl.BlockSpec((_IMGB, _NOBJ, 4), lambda b: (b, 0, 0),
                         memory_space=pltpu.SMEM),
            pl.BlockSpec((_IMGB, _NOBJ, 2), lambda b: (b, 0, 0),
                         memory_space=pltpu.SMEM),
            pl.BlockSpec((_IMGB, _NCAND), lambda b: (b, 0)),
            pl.BlockSpec((_IMGB, _NCAND), lambda b: (b, 0)),
        ],
        out_specs=pl.BlockSpec((1, _LANES), lambda b: (0, 0)),
        out_shape=jax.ShapeDtypeStruct((1, _LANES), jnp.float32),
    )(priorsb, priors8, boxes, locs8, scores8, cank, cani)

    n_pos_total = out[0, 0]
    loc_loss = out[0, 1] / (n_pos_total * 4.0)
    conf_loss = (out[0, 2] + out[0, 3]) / n_pos_total
    return conf_loss + loc_loss


# speedup vs baseline: 124.9178x; 1.0127x over previous
import functools

import jax
import jax.numpy as jnp
from jax import lax
from jax.experimental import pallas as pl
from jax.experimental.pallas import tpu as pltpu
from jax.experimental.pallas import tpu_sc as plsc

_NOBJ = 8
_THRESHOLD = 0.5
_NEG_POS_RATIO = 3
_NEG_INF = -1e30
_INIT_KEY = -3e30
_ROWS = 69
_LANES = 128
_IMGB = 16
_NPAD = _ROWS * _LANES
_SEGS = _NPAD // 16
_NCAND = 32
_IMGS_PER_TILE = 4


_STAGES_FULL = [(2, 1), (4, 2), (4, 1), (8, 4), (8, 2), (8, 1),
                (16, 8), (16, 4), (16, 2), (16, 1)]
_STAGES_MERGE = [(16, 8), (16, 4), (16, 2), (16, 1)]


def _mk_stage_consts(lane, stages):
    out = []
    for (k, j) in stages:
        p = lax.bitwise_xor(lane, j)
        low = jnp.where((lane & j) == 0, 1, 0)
        dirmax = jnp.where((lane & k) == 0, 1, 0)
        keep_max = 1 - lax.bitwise_xor(low, dirmax)
        out.append((p, keep_max.astype(jnp.float32)))
    return out


def _net(k_, v_, stage_consts):
    for (p, km) in stage_consts:
        pk = k_[p]
        pv = v_[p]
        agef = jnp.where(k_ >= pk, 1.0, 0.0)
        sel_a = km == agef
        k_ = jnp.where(sel_a, k_, pk)
        v_ = jnp.where(sel_a, v_, pv)
    return k_, v_


def _halves(rev, ak, av, bk, bv):
    rbk = bk[rev]
    rbv = bv[rev]
    sel = ak >= rbk
    hik = jnp.where(sel, ak, rbk)
    hiv = jnp.where(sel, av, rbv)
    lok = jnp.where(sel, rbk, ak)
    lov = jnp.where(sel, rbv, av)
    return hik, hiv, lok, lov


def _mine_kernel(s0_hbm, s1_hbm, outk_hbm, outi_hbm, b0, b1, db, dbi,
                 okv, oiv):
    cid = lax.axis_index("c")
    sid = lax.axis_index("s")
    wid = sid * 2 + cid
    lane = lax.broadcasted_iota(jnp.int32, (16,), 0)
    rev = 15 - lane
    sf = _mk_stage_consts(lane, _STAGES_FULL)
    sm = _mk_stage_consts(lane, _STAGES_MERGE)
    tperm = [lax.bitwise_xor(lane, sh) for sh in (8, 4, 2, 1)]

    def merge_c(t1k, t1v, t2k, t2v, ck, cv):
        sck, scv = _net(ck, cv, sf)
        uk, uv, ulk, ulv = _halves(rev, t2k, t2v, sck, scv)
        uk, uv = _net(uk, uv, sm)
        ulk, ulv = _net(ulk, ulv, sm)
        nt1k, nt1v, wk, wv = _halves(rev, t1k, t1v, uk, uv)
        nt1k, nt1v = _net(nt1k, nt1v, sm)
        wk, wv = _net(wk, wv, sm)
        nt2k, nt2v, _, _ = _halves(rev, wk, wv, ulk, ulv)
        nt2k, nt2v = _net(nt2k, nt2v, sm)
        return nt1k, nt1v, nt2k, nt2v

    for g in range(_IMGS_PER_TILE):
        img = wid * _IMGS_PER_TILE + g
        pltpu.sync_copy(s0_hbm.at[pl.ds(img * _NPAD, _NPAD)], b0)
        pltpu.sync_copy(s1_hbm.at[pl.ds(img * _NPAD, _NPAD)], b1)
        tail = b1[pl.ds(8720, 16)]
        lanef = lane.astype(jnp.float32)
        b1[pl.ds(8720, 16)] = jnp.where(lanef < 11.5, tail, _NEG_INF)
        for t in range(546, _SEGS):
            b1[pl.ds(t * 16, 16)] = jnp.full((16,), _NEG_INF, jnp.float32)

        def pass1(j, carry):
            m1, m2 = carry
            base = j * 16
            c = b1[pl.ds(base, 16)] - b0[pl.ds(base, 16)]
            db[pl.ds(base, 16)] = c
            m2n = jnp.maximum(m2, jnp.minimum(m1, c))
            m1n = jnp.maximum(m1, c)
            return m1n, m2n

        m1, m2 = lax.fori_loop(
            0, _SEGS, pass1,
            (jnp.full((16,), _INIT_KEY, jnp.float32),
             jnp.full((16,), _INIT_KEY, jnp.float32)))
        tv = jnp.minimum(m1, m2)
        for p in tperm:
            tv = jnp.minimum(tv, tv[p])
        thr = tv[0]

        def pass2b(j, off):
            base = j * 16
            c = db[pl.ds(base, 16)]
            okv2_base = off * 16
            dbi[pl.ds(okv2_base, 16)] = (lane + base).astype(jnp.float32)
            b0[pl.ds(okv2_base, 16)] = c
            mx = c
            for p in tperm:
                mx = jnp.maximum(mx, mx[p])
            return off + jnp.where(mx[0] >= thr, 1, 0)

        nsv = lax.fori_loop(0, _SEGS, pass2b, jnp.int32(0))

        def fin(j, carry):
            t1k, t1v, t2k, t2v = carry
            base = j * 16
            ck = b0[pl.ds(base, 16)]
            cv = dbi[pl.ds(base, 16)]
            return merge_c(t1k, t1v, t2k, t2v, ck, cv)

        init = (jnp.full((16,), _INIT_KEY, jnp.float32),
                jnp.zeros((16,), jnp.float32),
                jnp.full((16,), _INIT_KEY, jnp.float32),
                jnp.zeros((16,), jnp.float32))
        t1k, t1v, t2k, t2v = lax.fori_loop(0, nsv, fin, init)

        okv[pl.ds(0, 16)] = t1k
        okv[pl.ds(16, 16)] = t2k
        oiv[pl.ds(0, 16)] = t1v.astype(jnp.int32)
        oiv[pl.ds(16, 16)] = t2v.astype(jnp.int32)
        pltpu.sync_copy(okv, outk_hbm.at[pl.ds(img * _NCAND, _NCAND)])
        pltpu.sync_copy(oiv, outi_hbm.at[pl.ds(img * _NCAND, _NCAND)])


def _mine(s0f, s1f, batch):
    mesh = plsc.VectorSubcoreMesh(core_axis_name="c", subcore_axis_name="s")
    f = pl.kernel(
        _mine_kernel,
        out_type=[
            jax.ShapeDtypeStruct((batch * _NCAND,), jnp.float32),
            jax.ShapeDtypeStruct((batch * _NCAND,), jnp.int32),
        ],
        mesh=mesh,
        scratch_types=[
            pltpu.VMEM((_NPAD,), jnp.float32),
            pltpu.VMEM((_NPAD,), jnp.float32),
            pltpu.VMEM((_NPAD,), jnp.float32),
            pltpu.VMEM((_NPAD,), jnp.float32),
            pltpu.VMEM((_NCAND,), jnp.float32),
            pltpu.VMEM((_NCAND,), jnp.int32),
        ],
    )
    return f(s0f, s1f)


def _one_image(i, boxes_ref, locs8_ref, scores8_ref, cank_ref, cani_ref,
               prior_geom, iotas):
    (px0, py0, px1, py1, parea, pcx_l, pcy_l, pw_l, ph_l) = prior_geom
    (row, col, flat, valid, lcol, lcol32) = iotas

    m = jnp.full((_ROWS, _LANES), -1.0, dtype=jnp.float32)
    amax = jnp.zeros((_ROWS, _LANES), dtype=jnp.float32)
    pfe = []
    for j in range(_NOBJ):
        bcx = boxes_ref[i, j, 0]
        bcy = boxes_ref[i, j, 1]
        bw = boxes_ref[i, j, 2]
        bh = boxes_ref[i, j, 3]
        bx0 = bcx - bw * 0.5
        by0 = bcy - bh * 0.5
        bx1 = bcx + bw * 0.5
        by1 = bcy + bh * 0.5
        barea = (bx1 - bx0) * (by1 - by0)
        iw = jnp.maximum(jnp.minimum(px1, bx1) - jnp.maximum(px0, bx0), 0.0)
        ih = jnp.maximum(jnp.minimum(py1, by1) - jnp.maximum(py0, by0), 0.0)
        inter = iw * ih
        iou = inter / (parea + barea - inter)
        upd = iou > m
        amax = jnp.where(upd, float(j), amax)
        m = jnp.where(upd, iou, m)
        mx = jnp.max(jnp.where(valid, iou, -1.0))
        pfe.append(jnp.min(jnp.where(valid & (iou == mx), flat, 1e9)))

    for j in range(_NOBJ):
        hit = flat == pfe[j]
        amax = jnp.where(hit, float(j), amax)
        m = jnp.where(hit, 1.0, m)

    pres = [jnp.max(jnp.where(valid & (amax == float(j)), 1.0, 0.0))
            for j in range(_NOBJ)]

    o_lane = amax[0:1, :]
    m_lane = m[0:1, :]
    pres_lane = jnp.zeros((1, _LANES), dtype=jnp.float32)
    for j in range(_NOBJ):
        pres_lane = jnp.where(lcol == float(j), pres[j], pres_lane)
    pos_lane = (lcol < float(_NOBJ)) & (m_lane >= _THRESHOLD) \
        & (pres_lane > 0.5)
    n_pos = jnp.sum(jnp.where(pos_lane, 1.0, 0.0))

    bcx_l = jnp.ones((1, _LANES), dtype=jnp.float32)
    bcy_l = jnp.ones((1, _LANES), dtype=jnp.float32)
    bw_l = jnp.ones((1, _LANES), dtype=jnp.float32)
    bh_l = jnp.ones((1, _LANES), dtype=jnp.float32)
    for j in range(_NOBJ):
        sel = o_lane == float(j)
        bcx_l = jnp.where(sel, boxes_ref[i, j, 0], bcx_l)
        bcy_l = jnp.where(sel, boxes_ref[i, j, 1], bcy_l)
        bw_l = jnp.where(sel, boxes_ref[i, j, 2], bw_l)
        bh_l = jnp.where(sel, boxes_ref[i, j, 3], bh_l)

    lx = [jnp.zeros((1, _LANES), dtype=jnp.float32) for _ in range(4)]
    s0_l = jnp.zeros((1, _LANES), dtype=jnp.float32)
    s1_l = jnp.zeros((1, _LANES), dtype=jnp.float32)
    for p in range(_NOBJ):
        sel = lcol == float(p)
        for k in range(4):
            lx[k] = jnp.where(sel, locs8_ref[i, p, k], lx[k])
        s0_l = jnp.where(sel, scores8_ref[i, p, 0], s0_l)
        s1_l = jnp.where(sel, scores8_ref[i, p, 1], s1_l)

    gcx = (bcx_l - pcx_l) / (pw_l / 10.0)
    gcy = (bcy_l - pcy_l) / (ph_l / 10.0)
    gw = jnp.log(bw_l / pw_l) * 5.0
    gh = jnp.log(bh_l / ph_l) * 5.0
    loc_abs = (jnp.abs(lx[0] - gcx) + jnp.abs(lx[1] - gcy)
               + jnp.abs(lx[2] - gw) + jnp.abs(lx[3] - gh))
    loc_num = jnp.sum(jnp.where(pos_lane, loc_abs, 0.0))

    smax = jnp.maximum(s0_l, s1_l)
    smin = jnp.minimum(s0_l, s1_l)
    lse = smax + jnp.log(1.0 + jnp.exp(smin - smax))
    conf_pos = jnp.sum(jnp.where(pos_lane, lse - s1_l, 0.0))

    pw2 = jnp.zeros((1, _LANES), dtype=jnp.float32)
    for p in range(_NOBJ):
        pw2 = jnp.where(lcol == float(p), float(2 ** p), pw2)
    enc = jnp.sum(jnp.where(pos_lane, pw2, 0.0))

    keys = cank_ref[i:i + 1, :]
    idxf = cani_ref[i:i + 1, :].astype(jnp.float32)
    encv = jnp.zeros((1, _NCAND), jnp.float32) + enc
    p2 = jnp.exp2(jnp.minimum(idxf, 8.0))
    q = jnp.floor(encv / p2)
    bitv = q - jnp.floor(q * 0.5) * 2.0
    drop = (idxf < float(_NOBJ)) & (bitv > 0.5)
    surv = ~drop
    run = jnp.where(surv, 1.0, 0.0)
    for sh in (1, 2, 4, 8, 16):
        shifted = jnp.where(lcol32 >= float(sh),
                            pltpu.roll(run, sh, axis=1), 0.0)
        run = run + shifted
    pick = surv & (run <= float(_NEG_POS_RATIO) * n_pos)
    tmax = jnp.maximum(keys, 0.0)
    closs = tmax + jnp.log(1.0 + jnp.exp(-jnp.abs(keys)))
    hard_sum = jnp.sum(jnp.where(pick, closs, 0.0))

    return n_pos, loc_num, conf_pos, hard_sum


def _loss_kernel(priorsb_ref, priors8_ref, boxes_ref, locs8_ref, scores8_ref,
                 cank_ref, cani_ref, out_ref):
    b = pl.program_id(0)

    row_i = jax.lax.broadcasted_iota(jnp.int32, (_ROWS, _LANES), 0)
    col_i = jax.lax.broadcasted_iota(jnp.int32, (_ROWS, _LANES), 1)
    row = row_i.astype(jnp.float32)
    col = col_i.astype(jnp.float32)
    flat = (row_i * _LANES + col_i).astype(jnp.float32)
    valid = flat < 8732.0
    lcol = col[0:1, :]
    lcol32 = jax.lax.broadcasted_iota(jnp.int32, (1, _NCAND), 1) \
        .astype(jnp.float32)

    pcx = priorsb_ref[0]
    pcy = priorsb_ref[1]
    pw = priorsb_ref[2]
    ph = priorsb_ref[3]
    px0 = pcx - pw * 0.5
    py0 = pcy - ph * 0.5
    px1 = pcx + pw * 0.5
    py1 = pcy + ph * 0.5
    parea = (px1 - px0) * (py1 - py0)

    p8 = priors8_ref
    pcx_l = jnp.zeros((1, _LANES), dtype=jnp.float32)
    pcy_l = jnp.zeros((1, _LANES), dtype=jnp.float32)
    pw_l = jnp.ones((1, _LANES), dtype=jnp.float32)
    ph_l = jnp.ones((1, _LANES), dtype=jnp.float32)
    for p in range(_NOBJ):
        sel = lcol == float(p)
        pcx_l = jnp.where(sel, p8[p, 0], pcx_l)
        pcy_l = jnp.where(sel, p8[p, 1], pcy_l)
        pw_l = jnp.where(sel, p8[p, 2], pw_l)
        ph_l = jnp.where(sel, p8[p, 3], ph_l)

    prior_geom = (px0, py0, px1, py1, parea, pcx_l, pcy_l, pw_l, ph_l)
    iotas = (row, col, flat, valid, lcol, lcol32)

    n_pos_t = jnp.float32(0.0)
    loc_t = jnp.float32(0.0)
    cpos_t = jnp.float32(0.0)
    hard_t = jnp.float32(0.0)
    for i in range(_IMGB):
        n_pos, loc_num, conf_pos, hard_sum = _one_image(
            i, boxes_ref, locs8_ref, scores8_ref, cank_ref, cani_ref,
            prior_geom, iotas)
        n_pos_t += n_pos
        loc_t += loc_num
        cpos_t += conf_pos
        hard_t += hard_sum

    contrib = (jnp.where(lcol == 0.0, n_pos_t, 0.0)
               + jnp.where(lcol == 1.0, loc_t, 0.0)
               + jnp.where(lcol == 2.0, cpos_t, 0.0)
               + jnp.where(lcol == 3.0, hard_t, 0.0))

    @pl.when(b == 0)
    def _init():
        out_ref[...] = jnp.zeros_like(out_ref)

    out_ref[...] += contrib


def kernel(predicted_locs, predicted_scores, boxes, priors):
    batch, n_priors, _ = predicted_locs.shape
    pad = _NPAD - n_priors

    s0 = predicted_scores[..., 0]
    s1 = predicted_scores[..., 1]
    s0f = jnp.pad(s0, ((0, 0), (0, pad))).reshape(batch * _NPAD)
    s1f = jnp.pad(s1, ((0, 0), (0, pad))).reshape(batch * _NPAD)

    outk, outi = _mine(s0f, s1f, batch)
    cank = outk.reshape(batch, _NCAND)
    cani = outi.reshape(batch, _NCAND)

    pad_prior = jnp.tile(
        jnp.asarray([[-100.0, -100.0, 1.0, 1.0]], dtype=jnp.float32),
        (pad, 1))
    priorsb = jnp.concatenate([priors, pad_prior], axis=0).T.reshape(
        4, _ROWS, _LANES)
    priors8 = priors[:_NOBJ]
    locs8 = predicted_locs[:, :_NOBJ, :]
    scores8 = predicted_scores[:, :_NOBJ, :]

    out = pl.pallas_call(
        _loss_kernel,
        grid=(batch // _IMGB,),
        in_specs=[
            pl.BlockSpec((4, _ROWS, _LANES), lambda b: (0, 0, 0)),
            pl.BlockSpec((_NOBJ, 4), lambda b: (0, 0),
                         memory_space=pltpu.SMEM),
            pl.BlockSpec((_IMGB, _NOBJ, 4), lambda b: (b, 0, 0),
                         memory_space=pltpu.SMEM),
            pl.BlockSpec((_IMGB, _NOBJ, 4), lambda b: (b, 0, 0),
                         memory_space=pltpu.SMEM),
            pl.BlockSpec((_IMGB, _NOBJ, 2), lambda b: (b, 0, 0),
                         memory_space=pltpu.SMEM),
            pl.BlockSpec((_IMGB, _NCAND), lambda b: (b, 0)),
            pl.BlockSpec((_IMGB, _NCAND), lambda b: (b, 0)),
        ],
        out_specs=pl.BlockSpec((1, _LANES), lambda b: (0, 0)),
        out_shape=jax.ShapeDtypeStruct((1, _LANES), jnp.float32),
    )(priorsb, priors8, boxes, locs8, scores8, cank, cani)

    n_pos_total = out[0, 0]
    loc_loss = out[0, 1] / (n_pos_total * 4.0)
    conf_loss = (out[0, 2] + out[0, 3]) / n_pos_total
    return conf_loss + loc_loss


# speedup vs baseline: 289.6355x; 2.3186x over previous
import functools

import jax
import jax.numpy as jnp
from jax import lax
from jax.experimental import pallas as pl
from jax.experimental.pallas import tpu as pltpu
from jax.experimental.pallas import tpu_sc as plsc

_NOBJ = 8
_THRESHOLD = 0.5
_NEG_POS_RATIO = 3
_NEG_INF = -1e30
_INIT_KEY = -3e30
_ROWS = 69
_LANES = 128
_IMGB = 16
_NPAD = _ROWS * _LANES
_SEGS = _NPAD // 16
_NCAND = 32
_IMGS_PER_TILE = 4


_STAGES_FULL = [(2, 1), (4, 2), (4, 1), (8, 4), (8, 2), (8, 1),
                (16, 8), (16, 4), (16, 2), (16, 1)]
_STAGES_MERGE = [(16, 8), (16, 4), (16, 2), (16, 1)]


def _mk_stage_consts(lane, stages):
    out = []
    for (k, j) in stages:
        p = lax.bitwise_xor(lane, j)
        low = jnp.where((lane & j) == 0, 1, 0)
        dirmax = jnp.where((lane & k) == 0, 1, 0)
        keep_max = 1 - lax.bitwise_xor(low, dirmax)
        out.append((p, keep_max.astype(jnp.float32)))
    return out


def _net(k_, v_, stage_consts):
    for (p, km) in stage_consts:
        pk = k_[p]
        pv = v_[p]
        agef = jnp.where(k_ >= pk, 1.0, 0.0)
        sel_a = km == agef
        k_ = jnp.where(sel_a, k_, pk)
        v_ = jnp.where(sel_a, v_, pv)
    return k_, v_


def _halves(rev, ak, av, bk, bv):
    rbk = bk[rev]
    rbv = bv[rev]
    sel = ak >= rbk
    hik = jnp.where(sel, ak, rbk)
    hiv = jnp.where(sel, av, rbv)
    lok = jnp.where(sel, rbk, ak)
    lov = jnp.where(sel, rbv, av)
    return hik, hiv, lok, lov


def _mine_kernel(s0_hbm, s1_hbm, outk_hbm, outi_hbm, b0, b1, db, dbi,
                 okv, oiv):
    cid = lax.axis_index("c")
    sid = lax.axis_index("s")
    wid = sid * 2 + cid
    lane = lax.broadcasted_iota(jnp.int32, (16,), 0)
    rev = 15 - lane
    sf = _mk_stage_consts(lane, _STAGES_FULL)
    sm = _mk_stage_consts(lane, _STAGES_MERGE)
    tperm = [lax.bitwise_xor(lane, sh) for sh in (8, 4, 2, 1)]

    def merge_c(t1k, t1v, t2k, t2v, ck, cv):
        sck, scv = _net(ck, cv, sf)
        uk, uv, ulk, ulv = _halves(rev, t2k, t2v, sck, scv)
        uk, uv = _net(uk, uv, sm)
        ulk, ulv = _net(ulk, ulv, sm)
        nt1k, nt1v, wk, wv = _halves(rev, t1k, t1v, uk, uv)
        nt1k, nt1v = _net(nt1k, nt1v, sm)
        wk, wv = _net(wk, wv, sm)
        nt2k, nt2v, _, _ = _halves(rev, wk, wv, ulk, ulv)
        nt2k, nt2v = _net(nt2k, nt2v, sm)
        return nt1k, nt1v, nt2k, nt2v

    for g in range(_IMGS_PER_TILE):
        img = wid * _IMGS_PER_TILE + g
        pltpu.sync_copy(s0_hbm.at[pl.ds(img * _NPAD, _NPAD)], b0)
        pltpu.sync_copy(s1_hbm.at[pl.ds(img * _NPAD, _NPAD)], b1)
        tail = b1[pl.ds(8720, 16)]
        lanef = lane.astype(jnp.float32)
        b1[pl.ds(8720, 16)] = jnp.where(lanef < 11.5, tail, _NEG_INF)
        for t in range(546, _SEGS):
            b1[pl.ds(t * 16, 16)] = jnp.full((16,), _NEG_INF, jnp.float32)

        def pass1(j, carry):
            m1, m2 = carry
            base = j * 16
            c = b1[pl.ds(base, 16)] - b0[pl.ds(base, 16)]
            db[pl.ds(base, 16)] = c
            m2n = jnp.maximum(m2, jnp.minimum(m1, c))
            m1n = jnp.maximum(m1, c)
            return m1n, m2n

        m1, m2 = lax.fori_loop(
            0, _SEGS, pass1,
            (jnp.full((16,), _INIT_KEY, jnp.float32),
             jnp.full((16,), _INIT_KEY, jnp.float32)))
        tv = jnp.minimum(m1, m2)
        for p in tperm:
            tv = jnp.minimum(tv, tv[p])
        thr = tv[0]

        def pass2b(j, off):
            base = j * 16
            c = db[pl.ds(base, 16)]
            okv2_base = off * 16
            dbi[pl.ds(okv2_base, 16)] = (lane + base).astype(jnp.float32)
            b0[pl.ds(okv2_base, 16)] = c
            mx = c
            for p in tperm:
                mx = jnp.maximum(mx, mx[p])
            return off + jnp.where(mx[0] >= thr, 1, 0)

        nsv = lax.fori_loop(0, _SEGS, pass2b, jnp.int32(0))

        def fin(j, carry):
            t1k, t1v, t2k, t2v = carry
            base = j * 16
            ck = b0[pl.ds(base, 16)]
            cv = dbi[pl.ds(base, 16)]
            return merge_c(t1k, t1v, t2k, t2v, ck, cv)

        init = (jnp.full((16,), _INIT_KEY, jnp.float32),
                jnp.zeros((16,), jnp.float32),
                jnp.full((16,), _INIT_KEY, jnp.float32),
                jnp.zeros((16,), jnp.float32))
        t1k, t1v, t2k, t2v = lax.fori_loop(0, nsv, fin, init)

        okv[pl.ds(0, 16)] = t1k
        okv[pl.ds(16, 16)] = t2k
        oiv[pl.ds(0, 16)] = t1v.astype(jnp.int32)
        oiv[pl.ds(16, 16)] = t2v.astype(jnp.int32)
        pltpu.sync_copy(okv, outk_hbm.at[pl.ds(img * _NCAND, _NCAND)])
        pltpu.sync_copy(oiv, outi_hbm.at[pl.ds(img * _NCAND, _NCAND)])


def _mine(s0f, s1f, batch):
    mesh = plsc.VectorSubcoreMesh(core_axis_name="c", subcore_axis_name="s")
    f = pl.kernel(
        _mine_kernel,
        out_type=[
            jax.ShapeDtypeStruct((batch * _NCAND,), jnp.float32),
            jax.ShapeDtypeStruct((batch * _NCAND,), jnp.int32),
        ],
        mesh=mesh,
        scratch_types=[
            pltpu.VMEM((_NPAD,), jnp.float32),
            pltpu.VMEM((_NPAD,), jnp.float32),
            pltpu.VMEM((_NPAD,), jnp.float32),
            pltpu.VMEM((_NPAD,), jnp.float32),
            pltpu.VMEM((_NCAND,), jnp.float32),
            pltpu.VMEM((_NCAND,), jnp.int32),
        ],
    )
    return f(s0f, s1f)


def _box_iou(i, j, boxes_ref, prior_geom):
    (px0, py0, px1, py1, parea) = prior_geom[:5]
    bcx = boxes_ref[i, j, 0]
    bcy = boxes_ref[i, j, 1]
    bw = boxes_ref[i, j, 2]
    bh = boxes_ref[i, j, 3]
    bx0 = bcx - bw * 0.5
    by0 = bcy - bh * 0.5
    bx1 = bcx + bw * 0.5
    by1 = bcy + bh * 0.5
    barea = (bx1 - bx0) * (by1 - by0)
    iw = jnp.maximum(jnp.minimum(px1, bx1) - jnp.maximum(px0, bx0), 0.0)
    ih = jnp.maximum(jnp.minimum(py1, by1) - jnp.maximum(py0, by0), 0.0)
    inter = iw * ih
    return inter / (parea + barea - inter)


def _bcast(x11, shape):
    return jnp.broadcast_to(x11, shape)


def _loss_kernel(priorsb_ref, priors8_ref, boxes_ref, locs8_ref, scores8_ref,
                 cank_ref, cani_ref, out_ref):
    b = pl.program_id(0)

    row_i = jax.lax.broadcasted_iota(jnp.int32, (_ROWS, _LANES), 0)
    col_i = jax.lax.broadcasted_iota(jnp.int32, (_ROWS, _LANES), 1)
    col = col_i.astype(jnp.float32)
    flat = (row_i * _LANES + col_i).astype(jnp.float32)
    valid = flat < 8732.0
    lcol = col[0:1, :]
    lcol32 = jax.lax.broadcasted_iota(jnp.int32, (1, _NCAND), 1) \
        .astype(jnp.float32)

    pcx = priorsb_ref[0]
    pcy = priorsb_ref[1]
    pw = priorsb_ref[2]
    ph = priorsb_ref[3]
    px0 = pcx - pw * 0.5
    py0 = pcy - ph * 0.5
    px1 = pcx + pw * 0.5
    py1 = pcy + ph * 0.5
    parea = (px1 - px0) * (py1 - py0)

    p8 = priors8_ref
    pcx_l = jnp.zeros((1, _LANES), dtype=jnp.float32)
    pcy_l = jnp.zeros((1, _LANES), dtype=jnp.float32)
    pw_l = jnp.ones((1, _LANES), dtype=jnp.float32)
    ph_l = jnp.ones((1, _LANES), dtype=jnp.float32)
    pw2 = jnp.zeros((1, _LANES), dtype=jnp.float32)
    for p in range(_NOBJ):
        sel = lcol == float(p)
        pcx_l = jnp.where(sel, p8[p, 0], pcx_l)
        pcy_l = jnp.where(sel, p8[p, 1], pcy_l)
        pw_l = jnp.where(sel, p8[p, 2], pw_l)
        ph_l = jnp.where(sel, p8[p, 3], ph_l)
        pw2 = jnp.where(sel, float(2 ** p), pw2)

    geom = (px0, py0, px1, py1, parea)

    o_ls, m_ls, pres_ls = [], [], []
    bcx_ls, bcy_ls, bw_ls, bh_ls = [], [], [], []
    lx_ls = [[] for _ in range(4)]
    s0_ls, s1_ls = [], []
    for i in range(_IMGB):
        rms = []
        for j in range(_NOBJ):
            iou = _box_iou(i, j, boxes_ref, geom)
            rms.append(jnp.max(jnp.where(valid, iou, -1.0), axis=0,
                               keepdims=True))
        mx8 = jnp.max(jnp.concatenate(rms, axis=0), axis=1, keepdims=True)

        m = jnp.full((_ROWS, _LANES), -1.0, dtype=jnp.float32)
        amax = jnp.zeros((_ROWS, _LANES), dtype=jnp.float32)
        rmins = []
        for j in range(_NOBJ):
            iou = _box_iou(i, j, boxes_ref, geom)
            upd = iou > m
            amax = jnp.where(upd, float(j), amax)
            m = jnp.where(upd, iou, m)
            mxb = _bcast(mx8[j:j + 1, 0:1], (_ROWS, _LANES))
            cand = jnp.where((iou == mxb) & valid, flat, 1e9)
            rmins.append(jnp.min(cand, axis=0, keepdims=True))
        pfe8 = jnp.min(jnp.concatenate(rmins, axis=0), axis=1, keepdims=True)

        for j in range(_NOBJ):
            hit = flat == _bcast(pfe8[j:j + 1, 0:1], (_ROWS, _LANES))
            amax = jnp.where(hit, float(j), amax)
            m = jnp.where(hit, 1.0, m)

        rpres = []
        for j in range(_NOBJ):
            eq = jnp.where((amax == float(j)) & valid, 1.0, 0.0)
            rpres.append(jnp.max(eq, axis=0, keepdims=True))
        pres8 = jnp.max(jnp.concatenate(rpres, axis=0), axis=1, keepdims=True)

        o_lane = amax[0:1, :]
        pres_lane = jnp.zeros((1, _LANES), dtype=jnp.float32)
        bcx_l = jnp.ones((1, _LANES), dtype=jnp.float32)
        bcy_l = jnp.ones((1, _LANES), dtype=jnp.float32)
        bw_l = jnp.ones((1, _LANES), dtype=jnp.float32)
        bh_l = jnp.ones((1, _LANES), dtype=jnp.float32)
        for j in range(_NOBJ):
            pres_lane = jnp.where(lcol == float(j),
                                  _bcast(pres8[j:j + 1, 0:1], (1, _LANES)),
                                  pres_lane)
            sel = o_lane == float(j)
            bcx_l = jnp.where(sel, boxes_ref[i, j, 0], bcx_l)
            bcy_l = jnp.where(sel, boxes_ref[i, j, 1], bcy_l)
            bw_l = jnp.where(sel, boxes_ref[i, j, 2], bw_l)
            bh_l = jnp.where(sel, boxes_ref[i, j, 3], bh_l)
        lx = [jnp.zeros((1, _LANES), dtype=jnp.float32) for _ in range(4)]
        s0_l = jnp.zeros((1, _LANES), dtype=jnp.float32)
        s1_l = jnp.zeros((1, _LANES), dtype=jnp.float32)
        for p in range(_NOBJ):
            sel = lcol == float(p)
            for k in range(4):
                lx[k] = jnp.where(sel, locs8_ref[i, p, k], lx[k])
            s0_l = jnp.where(sel, scores8_ref[i, p, 0], s0_l)
            s1_l = jnp.where(sel, scores8_ref[i, p, 1], s1_l)

        o_ls.append(o_lane)
        m_ls.append(m[0:1, :])
        pres_ls.append(pres_lane)
        bcx_ls.append(bcx_l)
        bcy_ls.append(bcy_l)
        bw_ls.append(bw_l)
        bh_ls.append(bh_l)
        for k in range(4):
            lx_ls[k].append(lx[k])
        s0_ls.append(s0_l)
        s1_ls.append(s1_l)

    cat = lambda ls: jnp.concatenate(ls, axis=0)
    M0 = cat(m_ls)
    PRES = cat(pres_ls)
    POS = (lcol < float(_NOBJ)) & (M0 >= _THRESHOLD) & (PRES > 0.5)
    NPOS = jnp.sum(jnp.where(POS, 1.0, 0.0), axis=1, keepdims=True)

    GCX = (cat(bcx_ls) - pcx_l) / (pw_l / 10.0)
    GCY = (cat(bcy_ls) - pcy_l) / (ph_l / 10.0)
    GW = jnp.log(cat(bw_ls) / pw_l) * 5.0
    GH = jnp.log(cat(bh_ls) / ph_l) * 5.0
    labs = (jnp.abs(cat(lx_ls[0]) - GCX) + jnp.abs(cat(lx_ls[1]) - GCY)
            + jnp.abs(cat(lx_ls[2]) - GW) + jnp.abs(cat(lx_ls[3]) - GH))
    LOCN = jnp.sum(jnp.where(POS, labs, 0.0), axis=1, keepdims=True)

    S0 = cat(s0_ls)
    S1 = cat(s1_ls)
    smax = jnp.maximum(S0, S1)
    smin = jnp.minimum(S0, S1)
    lse = smax + jnp.log(1.0 + jnp.exp(smin - smax))
    CONF = jnp.sum(jnp.where(POS, lse - S1, 0.0), axis=1, keepdims=True)

    ENC = jnp.sum(jnp.where(POS, pw2, 0.0), axis=1, keepdims=True)

    keys = cank_ref[...]
    idxf = cani_ref[...].astype(jnp.float32)
    encv = ENC + keys * 0.0
    p2 = jnp.exp2(jnp.minimum(idxf, 8.0))
    q = jnp.floor(encv / p2)
    bitv = q - jnp.floor(q * 0.5) * 2.0
    drop = (idxf < float(_NOBJ)) & (bitv > 0.5)
    surv = ~drop
    run = jnp.where(surv, 1.0, 0.0)
    for sh in (1, 2, 4, 8, 16):
        shifted = jnp.where(lcol32 >= float(sh),
                            pltpu.roll(run, sh, axis=1), 0.0)
        run = run + shifted
    PICK = surv & (run <= float(_NEG_POS_RATIO) * NPOS)
    tmax = jnp.maximum(keys, 0.0)
    closs = tmax + jnp.log(1.0 + jnp.exp(-jnp.abs(keys)))
    HARD = jnp.sum(jnp.where(PICK, closs, 0.0), axis=1, keepdims=True)

    contrib = (jnp.where(lcol == 0.0, jnp.sum(NPOS), 0.0)
               + jnp.where(lcol == 1.0, jnp.sum(LOCN), 0.0)
               + jnp.where(lcol == 2.0, jnp.sum(CONF), 0.0)
               + jnp.where(lcol == 3.0, jnp.sum(HARD), 0.0))

    @pl.when(b == 0)
    def _init():
        out_ref[...] = jnp.zeros_like(out_ref)

    out_ref[...] += contrib


def kernel(predicted_locs, predicted_scores, boxes, priors):
    batch, n_priors, _ = predicted_locs.shape
    pad = _NPAD - n_priors

    s0 = predicted_scores[..., 0]
    s1 = predicted_scores[..., 1]
    s0f = jnp.pad(s0, ((0, 0), (0, pad))).reshape(batch * _NPAD)
    s1f = jnp.pad(s1, ((0, 0), (0, pad))).reshape(batch * _NPAD)

    outk, outi = _mine(s0f, s1f, batch)
    cank = outk.reshape(batch, _NCAND)
    cani = outi.reshape(batch, _NCAND)

    pad_prior = jnp.tile(
        jnp.asarray([[-100.0, -100.0, 1.0, 1.0]], dtype=jnp.float32),
        (pad, 1))
    priorsb = jnp.concatenate([priors, pad_prior], axis=0).T.reshape(
        4, _ROWS, _LANES)
    priors8 = priors[:_NOBJ]
    locs8 = predicted_locs[:, :_NOBJ, :]
    scores8 = predicted_scores[:, :_NOBJ, :]

    out = pl.pallas_call(
        _loss_kernel,
        grid=(batch // _IMGB,),
        in_specs=[
            pl.BlockSpec((4, _ROWS, _LANES), lambda b: (0, 0, 0)),
            pl.BlockSpec((_NOBJ, 4), lambda b: (0, 0),
                         memory_space=pltpu.SMEM),
            pl.BlockSpec((_IMGB, _NOBJ, 4), lambda b: (b, 0, 0),
                         memory_space=pltpu.SMEM),
            pl.BlockSpec((_IMGB, _NOBJ, 4), lambda b: (b, 0, 0),
                         memory_space=pltpu.SMEM),
            pl.BlockSpec((_IMGB, _NOBJ, 2), lambda b: (b, 0, 0),
                         memory_space=pltpu.SMEM),
            pl.BlockSpec((_IMGB, _NCAND), lambda b: (b, 0)),
            pl.BlockSpec((_IMGB, _NCAND), lambda b: (b, 0)),
        ],
        out_specs=pl.BlockSpec((1, _LANES), lambda b: (0, 0)),
        out_shape=jax.ShapeDtypeStruct((1, _LANES), jnp.float32),
    )(priorsb, priors8, boxes, locs8, scores8, cank, cani)

    n_pos_total = out[0, 0]
    loc_loss = out[0, 1] / (n_pos_total * 4.0)
    conf_loss = (out[0, 2] + out[0, 3]) / n_pos_total
    return conf_loss + loc_loss


# speedup vs baseline: 404.8027x; 1.3976x over previous
import functools

import jax
import jax.numpy as jnp
from jax import lax
from jax.experimental import pallas as pl
from jax.experimental.pallas import tpu as pltpu
from jax.experimental.pallas import tpu_sc as plsc

_NOBJ = 8
_THRESHOLD = 0.5
_NEG_POS_RATIO = 3
_NEG_INF = -1e30
_INIT_KEY = -3e30
_ROWS = 69
_LANES = 128
_IMGB = 16
_NPAD = _ROWS * _LANES
_SEGS = _NPAD // 16
_NCAND = 32
_IMGS_PER_TILE = 4


_STAGES_FULL = [(2, 1), (4, 2), (4, 1), (8, 4), (8, 2), (8, 1),
                (16, 8), (16, 4), (16, 2), (16, 1)]
_STAGES_MERGE = [(16, 8), (16, 4), (16, 2), (16, 1)]


def _mk_stage_consts(lane, stages):
    out = []
    for (k, j) in stages:
        p = lax.bitwise_xor(lane, j)
        low = jnp.where((lane & j) == 0, 1, 0)
        dirmax = jnp.where((lane & k) == 0, 1, 0)
        keep_max = 1 - lax.bitwise_xor(low, dirmax)
        out.append((p, keep_max.astype(jnp.float32)))
    return out


def _net(k_, v_, stage_consts):
    for (p, km) in stage_consts:
        pk = k_[p]
        pv = v_[p]
        agef = jnp.where(k_ >= pk, 1.0, 0.0)
        sel_a = km == agef
        k_ = jnp.where(sel_a, k_, pk)
        v_ = jnp.where(sel_a, v_, pv)
    return k_, v_


def _halves(rev, ak, av, bk, bv):
    rbk = bk[rev]
    rbv = bv[rev]
    sel = ak >= rbk
    hik = jnp.where(sel, ak, rbk)
    hiv = jnp.where(sel, av, rbv)
    lok = jnp.where(sel, rbk, ak)
    lov = jnp.where(sel, rbv, av)
    return hik, hiv, lok, lov


def _mine_kernel(s0_hbm, s1_hbm, outk_hbm, outi_hbm, b0, b1, db, dbi,
                 okv, oiv):
    cid = lax.axis_index("c")
    sid = lax.axis_index("s")
    wid = sid * 2 + cid
    lane = lax.broadcasted_iota(jnp.int32, (16,), 0)
    rev = 15 - lane
    sf = _mk_stage_consts(lane, _STAGES_FULL)
    sm = _mk_stage_consts(lane, _STAGES_MERGE)
    tperm = [lax.bitwise_xor(lane, sh) for sh in (8, 4, 2, 1)]

    def merge_c(t1k, t1v, t2k, t2v, ck, cv):
        sck, scv = _net(ck, cv, sf)
        uk, uv, ulk, ulv = _halves(rev, t2k, t2v, sck, scv)
        uk, uv = _net(uk, uv, sm)
        ulk, ulv = _net(ulk, ulv, sm)
        nt1k, nt1v, wk, wv = _halves(rev, t1k, t1v, uk, uv)
        nt1k, nt1v = _net(nt1k, nt1v, sm)
        wk, wv = _net(wk, wv, sm)
        nt2k, nt2v, _, _ = _halves(rev, wk, wv, ulk, ulv)
        nt2k, nt2v = _net(nt2k, nt2v, sm)
        return nt1k, nt1v, nt2k, nt2v

    for g in range(_IMGS_PER_TILE):
        img = wid * _IMGS_PER_TILE + g
        pltpu.sync_copy(s0_hbm.at[pl.ds(img * _NPAD, _NPAD)], b0)
        pltpu.sync_copy(s1_hbm.at[pl.ds(img * _NPAD, _NPAD)], b1)
        tail = b1[pl.ds(8720, 16)]
        lanef = lane.astype(jnp.float32)
        b1[pl.ds(8720, 16)] = jnp.where(lanef < 11.5, tail, _NEG_INF)
        for t in range(546, _SEGS):
            b1[pl.ds(t * 16, 16)] = jnp.full((16,), _NEG_INF, jnp.float32)

        def pass1(j, carry):
            m1, m2 = carry
            base = j * 16
            c = b1[pl.ds(base, 16)] - b0[pl.ds(base, 16)]
            db[pl.ds(base, 16)] = c
            m2n = jnp.maximum(m2, jnp.minimum(m1, c))
            m1n = jnp.maximum(m1, c)
            return m1n, m2n

        m1, m2 = lax.fori_loop(
            0, _SEGS, pass1,
            (jnp.full((16,), _INIT_KEY, jnp.float32),
             jnp.full((16,), _INIT_KEY, jnp.float32)))
        tv = jnp.minimum(m1, m2)
        for p in tperm:
            tv = jnp.minimum(tv, tv[p])
        thr = tv[0]

        def pass2b(j, off):
            base = j * 16
            c = db[pl.ds(base, 16)]
            okv2_base = off * 16
            dbi[pl.ds(okv2_base, 16)] = (lane + base).astype(jnp.float32)
            b0[pl.ds(okv2_base, 16)] = c
            mx = c
            for p in tperm:
                mx = jnp.maximum(mx, mx[p])
            return off + jnp.where(mx[0] >= thr, 1, 0)

        nsv = lax.fori_loop(0, _SEGS, pass2b, jnp.int32(0))

        def fin(j, carry):
            t1k, t1v, t2k, t2v = carry
            base = j * 16
            ck = b0[pl.ds(base, 16)]
            cv = dbi[pl.ds(base, 16)]
            return merge_c(t1k, t1v, t2k, t2v, ck, cv)

        init = (jnp.full((16,), _INIT_KEY, jnp.float32),
                jnp.zeros((16,), jnp.float32),
                jnp.full((16,), _INIT_KEY, jnp.float32),
                jnp.zeros((16,), jnp.float32))
        t1k, t1v, t2k, t2v = lax.fori_loop(0, nsv, fin, init)

        okv[pl.ds(0, 16)] = t1k
        okv[pl.ds(16, 16)] = t2k
        oiv[pl.ds(0, 16)] = t1v.astype(jnp.int32)
        oiv[pl.ds(16, 16)] = t2v.astype(jnp.int32)
        pltpu.sync_copy(okv, outk_hbm.at[pl.ds(img * _NCAND, _NCAND)])
        pltpu.sync_copy(oiv, outi_hbm.at[pl.ds(img * _NCAND, _NCAND)])


def _mine(s0f, s1f, batch):
    mesh = plsc.VectorSubcoreMesh(core_axis_name="c", subcore_axis_name="s")
    f = pl.kernel(
        _mine_kernel,
        out_type=[
            jax.ShapeDtypeStruct((batch * _NCAND,), jnp.float32),
            jax.ShapeDtypeStruct((batch * _NCAND,), jnp.int32),
        ],
        mesh=mesh,
        scratch_types=[
            pltpu.VMEM((_NPAD,), jnp.float32),
            pltpu.VMEM((_NPAD,), jnp.float32),
            pltpu.VMEM((_NPAD,), jnp.float32),
            pltpu.VMEM((_NPAD,), jnp.float32),
            pltpu.VMEM((_NCAND,), jnp.float32),
            pltpu.VMEM((_NCAND,), jnp.int32),
        ],
    )
    return f(s0f, s1f)


def _box_iou(i, j, boxes_ref, prior_geom):
    (px0, py0, px1, py1, parea) = prior_geom[:5]
    bcx = boxes_ref[i, j, 0]
    bcy = boxes_ref[i, j, 1]
    bw = boxes_ref[i, j, 2]
    bh = boxes_ref[i, j, 3]
    bx0 = bcx - bw * 0.5
    by0 = bcy - bh * 0.5
    bx1 = bcx + bw * 0.5
    by1 = bcy + bh * 0.5
    barea = (bx1 - bx0) * (by1 - by0)
    iw = jnp.maximum(jnp.minimum(px1, bx1) - jnp.maximum(px0, bx0), 0.0)
    ih = jnp.maximum(jnp.minimum(py1, by1) - jnp.maximum(py0, by0), 0.0)
    inter = iw * ih
    return inter / (parea + barea - inter)


def _bcast(x11, shape):
    return jnp.broadcast_to(x11, shape)


def _match_kernel(priorsb_ref, priors8_ref, boxes_ref, locs8_ref,
                  scores8_ref, out_ref):

    row_i = jax.lax.broadcasted_iota(jnp.int32, (_ROWS, _LANES), 0)
    col_i = jax.lax.broadcasted_iota(jnp.int32, (_ROWS, _LANES), 1)
    col = col_i.astype(jnp.float32)
    flat = (row_i * _LANES + col_i).astype(jnp.float32)
    valid = flat < 8732.0
    lcol = col[0:1, :]
    lcol32 = jax.lax.broadcasted_iota(jnp.int32, (1, _NCAND), 1) \
        .astype(jnp.float32)

    pcx = priorsb_ref[0]
    pcy = priorsb_ref[1]
    pw = priorsb_ref[2]
    ph = priorsb_ref[3]
    px0 = pcx - pw * 0.5
    py0 = pcy - ph * 0.5
    px1 = pcx + pw * 0.5
    py1 = pcy + ph * 0.5
    parea = (px1 - px0) * (py1 - py0)

    p8 = priors8_ref
    pcx_l = jnp.zeros((1, _LANES), dtype=jnp.float32)
    pcy_l = jnp.zeros((1, _LANES), dtype=jnp.float32)
    pw_l = jnp.ones((1, _LANES), dtype=jnp.float32)
    ph_l = jnp.ones((1, _LANES), dtype=jnp.float32)
    pw2 = jnp.zeros((1, _LANES), dtype=jnp.float32)
    for p in range(_NOBJ):
        sel = lcol == float(p)
        pcx_l = jnp.where(sel, p8[p, 0], pcx_l)
        pcy_l = jnp.where(sel, p8[p, 1], pcy_l)
        pw_l = jnp.where(sel, p8[p, 2], pw_l)
        ph_l = jnp.where(sel, p8[p, 3], ph_l)
        pw2 = jnp.where(sel, float(2 ** p), pw2)

    geom = (px0, py0, px1, py1, parea)

    o_ls, m_ls, pres_ls = [], [], []
    bcx_ls, bcy_ls, bw_ls, bh_ls = [], [], [], []
    lx_ls = [[] for _ in range(4)]
    s0_ls, s1_ls = [], []
    for i in range(_IMGB):
        rms = []
        for j in range(_NOBJ):
            iou = _box_iou(i, j, boxes_ref, geom)
            rms.append(jnp.max(jnp.where(valid, iou, -1.0), axis=0,
                               keepdims=True))
        mx8 = jnp.max(jnp.concatenate(rms, axis=0), axis=1, keepdims=True)

        m = jnp.full((_ROWS, _LANES), -1.0, dtype=jnp.float32)
        amax = jnp.zeros((_ROWS, _LANES), dtype=jnp.float32)
        rmins = []
        for j in range(_NOBJ):
            iou = _box_iou(i, j, boxes_ref, geom)
            upd = iou > m
            amax = jnp.where(upd, float(j), amax)
            m = jnp.where(upd, iou, m)
            mxb = _bcast(mx8[j:j + 1, 0:1], (_ROWS, _LANES))
            cand = jnp.where((iou == mxb) & valid, flat, 1e9)
            rmins.append(jnp.min(cand, axis=0, keepdims=True))
        pfe8 = jnp.min(jnp.concatenate(rmins, axis=0), axis=1, keepdims=True)

        for j in range(_NOBJ):
            hit = flat == _bcast(pfe8[j:j + 1, 0:1], (_ROWS, _LANES))
            amax = jnp.where(hit, float(j), amax)
            m = jnp.where(hit, 1.0, m)

        rpres = []
        for j in range(_NOBJ):
            eq = jnp.where((amax == float(j)) & valid, 1.0, 0.0)
            rpres.append(jnp.max(eq, axis=0, keepdims=True))
        pres8 = jnp.max(jnp.concatenate(rpres, axis=0), axis=1, keepdims=True)

        o_lane = amax[0:1, :]
        pres_lane = jnp.zeros((1, _LANES), dtype=jnp.float32)
        bcx_l = jnp.ones((1, _LANES), dtype=jnp.float32)
        bcy_l = jnp.ones((1, _LANES), dtype=jnp.float32)
        bw_l = jnp.ones((1, _LANES), dtype=jnp.float32)
        bh_l = jnp.ones((1, _LANES), dtype=jnp.float32)
        for j in range(_NOBJ):
            pres_lane = jnp.where(lcol == float(j),
                                  _bcast(pres8[j:j + 1, 0:1], (1, _LANES)),
                                  pres_lane)
            sel = o_lane == float(j)
            bcx_l = jnp.where(sel, boxes_ref[i, j, 0], bcx_l)
            bcy_l = jnp.where(sel, boxes_ref[i, j, 1], bcy_l)
            bw_l = jnp.where(sel, boxes_ref[i, j, 2], bw_l)
            bh_l = jnp.where(sel, boxes_ref[i, j, 3], bh_l)
        lx = [jnp.zeros((1, _LANES), dtype=jnp.float32) for _ in range(4)]
        s0_l = jnp.zeros((1, _LANES), dtype=jnp.float32)
        s1_l = jnp.zeros((1, _LANES), dtype=jnp.float32)
        for p in range(_NOBJ):
            sel = lcol == float(p)
            for k in range(4):
                lx[k] = jnp.where(sel, locs8_ref[i, p, k], lx[k])
            s0_l = jnp.where(sel, scores8_ref[i, p, 0], s0_l)
            s1_l = jnp.where(sel, scores8_ref[i, p, 1], s1_l)

        o_ls.append(o_lane)
        m_ls.append(m[0:1, :])
        pres_ls.append(pres_lane)
        bcx_ls.append(bcx_l)
        bcy_ls.append(bcy_l)
        bw_ls.append(bw_l)
        bh_ls.append(bh_l)
        for k in range(4):
            lx_ls[k].append(lx[k])
        s0_ls.append(s0_l)
        s1_ls.append(s1_l)

    cat = lambda ls: jnp.concatenate(ls, axis=0)
    M0 = cat(m_ls)
    PRES = cat(pres_ls)
    POS = (lcol < float(_NOBJ)) & (M0 >= _THRESHOLD) & (PRES > 0.5)
    NPOS = jnp.sum(jnp.where(POS, 1.0, 0.0), axis=1, keepdims=True)

    GCX = (cat(bcx_ls) - pcx_l) / (pw_l / 10.0)
    GCY = (cat(bcy_ls) - pcy_l) / (ph_l / 10.0)
    GW = jnp.log(cat(bw_ls) / pw_l) * 5.0
    GH = jnp.log(cat(bh_ls) / ph_l) * 5.0
    labs = (jnp.abs(cat(lx_ls[0]) - GCX) + jnp.abs(cat(lx_ls[1]) - GCY)
            + jnp.abs(cat(lx_ls[2]) - GW) + jnp.abs(cat(lx_ls[3]) - GH))
    LOCN = jnp.sum(jnp.where(POS, labs, 0.0), axis=1, keepdims=True)

    S0 = cat(s0_ls)
    S1 = cat(s1_ls)
    smax = jnp.maximum(S0, S1)
    smin = jnp.minimum(S0, S1)
    lse = smax + jnp.log(1.0 + jnp.exp(smin - smax))
    CONF = jnp.sum(jnp.where(POS, lse - S1, 0.0), axis=1, keepdims=True)

    ENC = jnp.sum(jnp.where(POS, pw2, 0.0), axis=1, keepdims=True)

    out_ref[...] = (jnp.where(lcol == 0.0, NPOS, 0.0)
                    + jnp.where(lcol == 1.0, LOCN, 0.0)
                    + jnp.where(lcol == 2.0, CONF, 0.0)
                    + jnp.where(lcol == 3.0, ENC, 0.0))


def _combine_kernel(stats_ref, cank_ref, cani_ref, out_ref):
    lcol = jax.lax.broadcasted_iota(jnp.int32, (1, _LANES), 1) \
        .astype(jnp.float32)
    lcol32 = jax.lax.broadcasted_iota(jnp.int32, (1, _NCAND), 1) \
        .astype(jnp.float32)
    stats = stats_ref[...]
    NPOS = stats[:, 0:1]
    LOCN = stats[:, 1:2]
    CONF = stats[:, 2:3]
    ENC = stats[:, 3:4]

    keys = cank_ref[...]
    idxf = cani_ref[...].astype(jnp.float32)
    encv = ENC + keys * 0.0
    p2 = jnp.exp2(jnp.minimum(idxf, 8.0))
    q = jnp.floor(encv / p2)
    bitv = q - jnp.floor(q * 0.5) * 2.0
    drop = (idxf < float(_NOBJ)) & (bitv > 0.5)
    surv = ~drop
    run = jnp.where(surv, 1.0, 0.0)
    for sh in (1, 2, 4, 8, 16):
        shifted = jnp.where(lcol32 >= float(sh),
                            pltpu.roll(run, sh, axis=1), 0.0)
        run = run + shifted
    PICK = surv & (run <= float(_NEG_POS_RATIO) * NPOS)
    tmax = jnp.maximum(keys, 0.0)
    closs = tmax + jnp.log(1.0 + jnp.exp(-jnp.abs(keys)))
    HARD = jnp.sum(jnp.where(PICK, closs, 0.0), axis=1, keepdims=True)

    out_ref[...] = (jnp.where(lcol == 0.0, jnp.sum(NPOS), 0.0)
                    + jnp.where(lcol == 1.0, jnp.sum(LOCN), 0.0)
                    + jnp.where(lcol == 2.0, jnp.sum(CONF), 0.0)
                    + jnp.where(lcol == 3.0, jnp.sum(HARD), 0.0))


def kernel(predicted_locs, predicted_scores, boxes, priors):
    batch, n_priors, _ = predicted_locs.shape
    pad = _NPAD - n_priors

    s0 = predicted_scores[..., 0]
    s1 = predicted_scores[..., 1]
    s0f = jnp.pad(s0, ((0, 0), (0, pad))).reshape(batch * _NPAD)
    s1f = jnp.pad(s1, ((0, 0), (0, pad))).reshape(batch * _NPAD)

    outk, outi = _mine(s0f, s1f, batch)
    cank = outk.reshape(batch, _NCAND)
    cani = outi.reshape(batch, _NCAND)

    pad_prior = jnp.tile(
        jnp.asarray([[-100.0, -100.0, 1.0, 1.0]], dtype=jnp.float32),
        (pad, 1))
    priorsb = jnp.concatenate([priors, pad_prior], axis=0).T.reshape(
        4, _ROWS, _LANES)
    priors8 = priors[:_NOBJ]
    locs8 = predicted_locs[:, :_NOBJ, :]
    scores8 = predicted_scores[:, :_NOBJ, :]

    stats = pl.pallas_call(
        _match_kernel,
        grid=(batch // _IMGB,),
        in_specs=[
            pl.BlockSpec((4, _ROWS, _LANES), lambda b: (0, 0, 0)),
            pl.BlockSpec((_NOBJ, 4), lambda b: (0, 0),
                         memory_space=pltpu.SMEM),
            pl.BlockSpec((_IMGB, _NOBJ, 4), lambda b: (b, 0, 0),
                         memory_space=pltpu.SMEM),
            pl.BlockSpec((_IMGB, _NOBJ, 4), lambda b: (b, 0, 0),
                         memory_space=pltpu.SMEM),
            pl.BlockSpec((_IMGB, _NOBJ, 2), lambda b: (b, 0, 0),
                         memory_space=pltpu.SMEM),
        ],
        out_specs=pl.BlockSpec((_IMGB, _LANES), lambda b: (b, 0)),
        out_shape=jax.ShapeDtypeStruct((batch, _LANES), jnp.float32),
    )(priorsb, priors8, boxes, locs8, scores8)

    out = pl.pallas_call(
        _combine_kernel,
        in_specs=[
            pl.BlockSpec((batch, _LANES), lambda: (0, 0)),
            pl.BlockSpec((batch, _NCAND), lambda: (0, 0)),
            pl.BlockSpec((batch, _NCAND), lambda: (0, 0)),
        ],
        out_specs=pl.BlockSpec((1, _LANES), lambda: (0, 0)),
        out_shape=jax.ShapeDtypeStruct((1, _LANES), jnp.float32),
    )(stats, cank, cani)

    n_pos_total = out[0, 0]
    loc_loss = out[0, 1] / (n_pos_total * 4.0)
    conf_loss = (out[0, 2] + out[0, 3]) / n_pos_total
    return conf_loss + loc_loss
